# Initial kernel scaffold; baseline (speedup 1.0000x reference)
import jax
import jax.numpy as jnp
from jax.experimental import pallas as pl


def kernel(data, user_factors, item_factors):
    raise NotImplementedError("write your pallas kernel here")


# speedup vs baseline: 4.4164x; 4.4164x over previous
import jax
import jax.numpy as jnp
from jax import lax
from jax.experimental import pallas as pl
from jax.experimental.pallas import tpu as pltpu
from jax.experimental.pallas import tpu_sc as plsc

_N = 16384
_L = 16
_U_PAD = 4512
_V_PAD = 6000

_INFO = plsc.get_sparse_core_info()
_NC = _INFO.num_cores
_NW = _NC * _INFO.num_subcores
_BPW = _N // _NW
_GROUPS = _BPW // _L


def _sc_body(user_hbm, item_hbm, u_hbm, v_hbm, out_hbm,
             uidx_v, iidx_v, u_v, v_v, out_v):
    wid = lax.axis_index("s") * _NC + lax.axis_index("c")
    base = wid * _BPW

    pltpu.sync_copy(u_hbm, u_v)
    pltpu.sync_copy(v_hbm, v_v)
    pltpu.sync_copy(user_hbm.at[pl.ds(base, _BPW)], uidx_v)
    pltpu.sync_copy(item_hbm.at[pl.ds(base, _BPW)], iidx_v)

    def body(g, carry):
        off = g * _L
        ua = uidx_v[pl.ds(off, _L)] * 3
        ia = iidx_v[pl.ds(off, _L)] * 3
        u0 = plsc.load_gather(u_v, [ua])
        u1 = plsc.load_gather(u_v, [ua + 1])
        u2 = plsc.load_gather(u_v, [ua + 2])
        w0 = plsc.load_gather(v_v, [ia])
        w1 = plsc.load_gather(v_v, [ia + 1])
        w2 = plsc.load_gather(v_v, [ia + 2])
        out_v[pl.ds(off, _L)] = u0 * w0 + u1 * w1 + u2 * w2
        return carry

    lax.fori_loop(0, _GROUPS, body, 0)
    pltpu.sync_copy(out_v, out_hbm.at[pl.ds(base, _BPW)])


def _sc_call(user, item, uflat, vflat):
    mesh = plsc.VectorSubcoreMesh(core_axis_name="c", subcore_axis_name="s")
    return pl.kernel(
        _sc_body,
        out_type=jax.ShapeDtypeStruct((_N,), jnp.float32),
        mesh=mesh,
        compiler_params=pltpu.CompilerParams(needs_layout_passes=False),
        scratch_types=[
            pltpu.VMEM((_BPW,), jnp.int32),
            pltpu.VMEM((_BPW,), jnp.int32),
            pltpu.VMEM((_U_PAD,), jnp.float32),
            pltpu.VMEM((_V_PAD,), jnp.float32),
            pltpu.VMEM((_BPW,), jnp.float32),
        ],
    )(user, item, uflat, vflat)


def kernel(data, user_factors, item_factors):
    data = data.astype(jnp.int32)
    user = data[0]
    item = data[1]
    uflat = jnp.pad(user_factors.reshape(-1), (0, _U_PAD - 4500))
    vflat = jnp.pad(item_factors.reshape(-1), (0, _V_PAD - 6000))
    return _sc_call(user, item, uflat, vflat)


# speedup vs baseline: 4.5384x; 1.0276x over previous
import jax
import jax.numpy as jnp
from jax import lax
from jax.experimental import pallas as pl
from jax.experimental.pallas import tpu as pltpu
from jax.experimental.pallas import tpu_sc as plsc

_N = 16384
_L = 16
_NU = 1500
_NV = 2000

_INFO = plsc.get_sparse_core_info()
_NC = _INFO.num_cores
_NW = _NC * _INFO.num_subcores
_BPW = _N // _NW
_GROUPS = _BPW // _L


def _sc_body(data_hbm, u_hbm, v_hbm, out_hbm,
             uidx_v, iidx_v, u_v, v_v, out_v, sem):
    wid = lax.axis_index("s") * _NC + lax.axis_index("c")
    base = wid * _BPW

    cps = [
        pltpu.async_copy(u_hbm, u_v, sem),
        pltpu.async_copy(v_hbm, v_v, sem),
        pltpu.async_copy(data_hbm.at[0, pl.ds(base, _BPW)], uidx_v, sem),
        pltpu.async_copy(data_hbm.at[1, pl.ds(base, _BPW)], iidx_v, sem),
    ]
    for cp in cps:
        cp.wait()

    c0 = jnp.zeros((_L,), jnp.int32)
    c1 = jnp.full((_L,), 1, jnp.int32)
    c2 = jnp.full((_L,), 2, jnp.int32)
    for g in range(_GROUPS):
        off = g * _L
        ui = uidx_v[pl.ds(off, _L)]
        ii = iidx_v[pl.ds(off, _L)]
        u0 = plsc.load_gather(u_v, [ui, c0])
        u1 = plsc.load_gather(u_v, [ui, c1])
        u2 = plsc.load_gather(u_v, [ui, c2])
        w0 = plsc.load_gather(v_v, [ii, c0])
        w1 = plsc.load_gather(v_v, [ii, c1])
        w2 = plsc.load_gather(v_v, [ii, c2])
        out_v[pl.ds(off, _L)] = u0 * w0 + u1 * w1 + u2 * w2

    pltpu.sync_copy(out_v, out_hbm.at[pl.ds(base, _BPW)])


def kernel(data, user_factors, item_factors):
    data = data.astype(jnp.int32)
    mesh = plsc.VectorSubcoreMesh(core_axis_name="c", subcore_axis_name="s")
    return pl.kernel(
        _sc_body,
        out_type=jax.ShapeDtypeStruct((_N,), jnp.float32),
        mesh=mesh,
        compiler_params=pltpu.CompilerParams(
            needs_layout_passes=False, use_tc_tiling_on_sc=False),
        scratch_types=[
            pltpu.VMEM((_BPW,), jnp.int32),
            pltpu.VMEM((_BPW,), jnp.int32),
            pltpu.VMEM((_NU, 3), jnp.float32),
            pltpu.VMEM((_NV, 3), jnp.float32),
            pltpu.VMEM((_BPW,), jnp.float32),
            pltpu.SemaphoreType.DMA,
        ],
    )(data, user_factors, item_factors)


# speedup vs baseline: 5.0497x; 1.1127x over previous
import jax
import jax.numpy as jnp
from jax import lax
from jax.experimental import pallas as pl
from jax.experimental.pallas import tpu as pltpu
from jax.experimental.pallas import tpu_sc as plsc

_N = 16384
_L = 16
_NU = 1500
_NV = 2000

_NC = 1
_NS = 16
_NW = _NC * _NS
_BPW = _N // _NW
_GROUPS = _BPW // _L


def _sc_body(data_hbm, u_hbm, v_hbm, out_hbm,
             uidx_v, iidx_v, u_v, v_v, out_v, sem):
    wid = lax.axis_index("s") * _NC + lax.axis_index("c")
    base = wid * _BPW

    cps = [
        pltpu.async_copy(u_hbm, u_v, sem),
        pltpu.async_copy(v_hbm, v_v, sem),
        pltpu.async_copy(data_hbm.at[0, pl.ds(base, _BPW)], uidx_v, sem),
        pltpu.async_copy(data_hbm.at[1, pl.ds(base, _BPW)], iidx_v, sem),
    ]
    for cp in cps:
        cp.wait()

    c0 = jnp.zeros((_L,), jnp.int32)
    c1 = jnp.full((_L,), 1, jnp.int32)
    c2 = jnp.full((_L,), 2, jnp.int32)
    for g in range(_GROUPS):
        off = g * _L
        ui = uidx_v[pl.ds(off, _L)]
        ii = iidx_v[pl.ds(off, _L)]
        u0 = plsc.load_gather(u_v, [ui, c0])
        u1 = plsc.load_gather(u_v, [ui, c1])
        u2 = plsc.load_gather(u_v, [ui, c2])
        w0 = plsc.load_gather(v_v, [ii, c0])
        w1 = plsc.load_gather(v_v, [ii, c1])
        w2 = plsc.load_gather(v_v, [ii, c2])
        out_v[pl.ds(off, _L)] = u0 * w0 + u1 * w1 + u2 * w2

    pltpu.sync_copy(out_v, out_hbm.at[pl.ds(base, _BPW)])


def kernel(data, user_factors, item_factors):
    data = data.astype(jnp.int32)
    mesh = plsc.VectorSubcoreMesh(
        core_axis_name="c", subcore_axis_name="s",
        num_cores=_NC, num_subcores=_NS)
    return pl.kernel(
        _sc_body,
        out_type=jax.ShapeDtypeStruct((_N,), jnp.float32),
        mesh=mesh,
        compiler_params=pltpu.CompilerParams(
            needs_layout_passes=False, use_tc_tiling_on_sc=False),
        scratch_types=[
            pltpu.VMEM((_BPW,), jnp.int32),
            pltpu.VMEM((_BPW,), jnp.int32),
            pltpu.VMEM((_NU, 3), jnp.float32),
            pltpu.VMEM((_NV, 3), jnp.float32),
            pltpu.VMEM((_BPW,), jnp.float32),
            pltpu.SemaphoreType.DMA,
        ],
    )(data, user_factors, item_factors)


# speedup vs baseline: 5.2175x; 1.0332x over previous
import jax
import jax.numpy as jnp
from jax import lax
from jax.experimental import pallas as pl
from jax.experimental.pallas import tpu as pltpu
from jax.experimental.pallas import tpu_sc as plsc

_N = 16384
_L = 16
_NU = 1500
_NV = 2000

_NC = 1
_NS = 16
_NW = _NC * _NS
_BPW = _N // _NW
_GROUPS = _BPW // _L


def _sc_body(data_hbm, u_hbm, v_hbm, out_hbm,
             uidx_v, iidx_v, u_v, v_v, out_v, sem):
    wid = lax.axis_index("s") * _NC + lax.axis_index("c")
    base = wid * _BPW

    cps = [
        pltpu.async_copy(u_hbm, u_v, sem),
        pltpu.async_copy(v_hbm, v_v, sem),
        pltpu.async_copy(data_hbm.at[0, pl.ds(base, _BPW)], uidx_v, sem),
        pltpu.async_copy(data_hbm.at[1, pl.ds(base, _BPW)], iidx_v, sem),
    ]
    for cp in cps:
        cp.wait()

    c0 = jnp.zeros((_L,), jnp.int32)
    c1 = jnp.full((_L,), 1, jnp.int32)
    c2 = jnp.full((_L,), 2, jnp.int32)
    def body(g, carry):
        off = g * _L
        ui = uidx_v[pl.ds(off, _L)]
        ii = iidx_v[pl.ds(off, _L)]
        u0 = plsc.load_gather(u_v, [ui, c0])
        u1 = plsc.load_gather(u_v, [ui, c1])
        u2 = plsc.load_gather(u_v, [ui, c2])
        w0 = plsc.load_gather(v_v, [ii, c0])
        w1 = plsc.load_gather(v_v, [ii, c1])
        w2 = plsc.load_gather(v_v, [ii, c2])
        out_v[pl.ds(off, _L)] = u0 * w0 + u1 * w1 + u2 * w2
        return carry

    lax.fori_loop(0, _GROUPS, body, 0)

    pltpu.sync_copy(out_v, out_hbm.at[pl.ds(base, _BPW)])


def kernel(data, user_factors, item_factors):
    data = data.astype(jnp.int32)
    mesh = plsc.VectorSubcoreMesh(
        core_axis_name="c", subcore_axis_name="s",
        num_cores=_NC, num_subcores=_NS)
    return pl.kernel(
        _sc_body,
        out_type=jax.ShapeDtypeStruct((_N,), jnp.float32),
        mesh=mesh,
        compiler_params=pltpu.CompilerParams(
            needs_layout_passes=False, use_tc_tiling_on_sc=False),
        scratch_types=[
            pltpu.VMEM((_BPW,), jnp.int32),
            pltpu.VMEM((_BPW,), jnp.int32),
            pltpu.VMEM((_NU, 3), jnp.float32),
            pltpu.VMEM((_NV, 3), jnp.float32),
            pltpu.VMEM((_BPW,), jnp.float32),
            pltpu.SemaphoreType.DMA,
        ],
    )(data, user_factors, item_factors)


# speedup vs baseline: 5.2760x; 1.0112x over previous
import jax
import jax.numpy as jnp
from jax import lax
from jax.experimental import pallas as pl
from jax.experimental.pallas import tpu as pltpu
from jax.experimental.pallas import tpu_sc as plsc

_N = 16384
_L = 16
_NU = 1500
_NV = 2000

_NC = 1
_NS = 16
_NW = _NC * _NS
_BPW = _N // _NW
_GROUPS = _BPW // _L


def _sc_body(data_hbm, u_hbm, v_hbm, out_hbm,
             uidx_v, iidx_v, u_v, v_v, out_v, sem):
    wid = lax.axis_index("s") * _NC + lax.axis_index("c")
    base = wid * _BPW

    cps = [
        pltpu.async_copy(u_hbm, u_v, sem),
        pltpu.async_copy(v_hbm, v_v, sem),
        pltpu.async_copy(data_hbm.at[0, pl.ds(base, _BPW)], uidx_v, sem),
        pltpu.async_copy(data_hbm.at[1, pl.ds(base, _BPW)], iidx_v, sem),
    ]
    for cp in cps:
        cp.wait()

    c0 = jnp.zeros((_L,), jnp.int32)
    c1 = jnp.full((_L,), 1, jnp.int32)
    c2 = jnp.full((_L,), 2, jnp.int32)
    @plsc.parallel_loop(0, _BPW, step=_L, unroll=4)
    def body(off):
        ui = uidx_v[pl.ds(off, _L)]
        ii = iidx_v[pl.ds(off, _L)]
        u0 = plsc.load_gather(u_v, [ui, c0])
        u1 = plsc.load_gather(u_v, [ui, c1])
        u2 = plsc.load_gather(u_v, [ui, c2])
        w0 = plsc.load_gather(v_v, [ii, c0])
        w1 = plsc.load_gather(v_v, [ii, c1])
        w2 = plsc.load_gather(v_v, [ii, c2])
        out_v[pl.ds(off, _L)] = u0 * w0 + u1 * w1 + u2 * w2

    pltpu.sync_copy(out_v, out_hbm.at[pl.ds(base, _BPW)])


def kernel(data, user_factors, item_factors):
    data = data.astype(jnp.int32)
    mesh = plsc.VectorSubcoreMesh(
        core_axis_name="c", subcore_axis_name="s",
        num_cores=_NC, num_subcores=_NS)
    return pl.kernel(
        _sc_body,
        out_type=jax.ShapeDtypeStruct((_N,), jnp.float32),
        mesh=mesh,
        compiler_params=pltpu.CompilerParams(
            needs_layout_passes=False, use_tc_tiling_on_sc=False),
        scratch_types=[
            pltpu.VMEM((_BPW,), jnp.int32),
            pltpu.VMEM((_BPW,), jnp.int32),
            pltpu.VMEM((_NU, 3), jnp.float32),
            pltpu.VMEM((_NV, 3), jnp.float32),
            pltpu.VMEM((_BPW,), jnp.float32),
            pltpu.SemaphoreType.DMA,
        ],
    )(data, user_factors, item_factors)
